# baseline (speedup 1.0000x reference)
import functools

import jax
import jax.numpy as jnp
from jax import lax
from jax.experimental import pallas as pl
from jax.experimental.pallas import tpu as pltpu
from jax.experimental.pallas import tpu_sc as plsc

_NW = 32
_C = 128
_NBUF = 10
_NT = 4
_L = 16
_PITCH = 129


def _make_gather(n_total, seq, batch, vocab, embed):
    per_w = n_total // _NW
    nch = per_w // _C
    ngrp = nch // _NBUF
    ntile = embed // 8
    mesh = plsc.VectorSubcoreMesh(core_axis_name="c", subcore_axis_name="s")

    @functools.partial(
        pl.kernel,
        mesh=mesh,
        out_type=jax.ShapeDtypeStruct((seq, ntile, batch // _C, 8, _C),
                                      jnp.float32),
        scratch_types=(
            [pltpu.VMEM((nch, _C), jnp.int32),
             pltpu.VMEM((_NBUF, _C, embed), jnp.float32),
             pltpu.VMEM((_NT, ntile, 8, _PITCH), jnp.float32)]
            + [pltpu.SemaphoreType.DMA] * (_NBUF + _NT)
        ),
        compiler_params=pltpu.CompilerParams(use_tc_tiling_on_sc=False,
                                             needs_layout_passes=False),
    )
    def k(q_hbm, table_hbm, out_hbm, idx_v, rows_v, gt_v, *sems):
        gsems, ssems = sems[:_NBUF], sems[_NBUF:]
        wid = lax.axis_index("s") * 2 + lax.axis_index("c")
        base = wid * per_w
        pltpu.sync_copy(q_hbm.at[wid], idx_v)

        iota = lax.iota(jnp.int32, _L)
        ei_vec = lax.bitwise_and(iota, 7)
        et_vecs = [lax.shift_right_logical(iota, 3) + 2 * kk
                   for kk in range(embed // _L)]

        def gather_start(i, b):
            return pltpu.async_copy(
                table_hbm.at[idx_v.at[i]], rows_v.at[b], gsems[b])

        def out_wait(t):
            pltpu.make_async_copy(
                gt_v.at[t, :, :, pl.ds(0, _C)],
                out_hbm.at[0, :, 0, :, :], ssems[t]).wait()

        for b in range(_NBUF):
            gather_start(b, b)

        def group(g, carry):
            for b in range(_NBUF):
                i = g * _NBUF + b
                f = base + i * _C
                s_idx = f // batch
                bt_idx = (f % batch) // _C
                t = b % _NT
                if b >= _NT:
                    out_wait(t)
                else:
                    @pl.when(g > 0)
                    def _():
                        out_wait(t)
                pltpu.make_async_copy(
                    table_hbm.at[idx_v.at[i]], rows_v.at[b], gsems[b]).wait()

                grow = rows_v.at[b]

                @plsc.parallel_loop(0, _C, unroll=16)
                def tr_body(bi):
                    bi_vec = jnp.full((_L,), 0, jnp.int32) + bi
                    for kk in range(embed // _L):
                        x = grow[bi, pl.ds(kk * _L, _L)]
                        plsc.store_scatter(
                            gt_v.at[t], [et_vecs[kk], ei_vec, bi_vec], x)

                pltpu.async_copy(
                    gt_v.at[t, :, :, pl.ds(0, _C)],
                    out_hbm.at[s_idx, :, bt_idx, :, :], ssems[t])

                @pl.when(g < ngrp - 1)
                def _():
                    gather_start(i + _NBUF, b)
            return carry

        lax.fori_loop(0, ngrp, group, 0)
        for t in range(_NT):
            out_wait(t)

    return k


def kernel(q, q_len, table):
    del q_len
    n_total = q.size
    batch, seq = q.shape
    vocab, embed = table.shape
    qt_blocked = q.T.reshape(_NW, n_total // (_NW * _C), _C).astype(jnp.int32)
    out = _make_gather(n_total, seq, batch, vocab, embed)(qt_blocked, table)
    return out.transpose(2, 4, 0, 1, 3).reshape(batch, seq, embed)


# speedup vs baseline: 1.0127x; 1.0127x over previous
import functools

import jax
import jax.numpy as jnp
from jax import lax
from jax.experimental import pallas as pl
from jax.experimental.pallas import tpu as pltpu
from jax.experimental.pallas import tpu_sc as plsc

_NW = 32
_C = 128
_NBUF = 8
_NT = 4
_L = 16
_PITCH = 129


def _make_gather(n_total, seq, batch, vocab, embed):
    per_w = n_total // _NW
    nch = per_w // _C
    ngrp = nch // _NBUF
    ntile = embed // 8
    mesh = plsc.VectorSubcoreMesh(core_axis_name="c", subcore_axis_name="s")

    @functools.partial(
        pl.kernel,
        mesh=mesh,
        out_type=jax.ShapeDtypeStruct((seq, ntile, batch // _C, 8, _C),
                                      jnp.float32),
        scratch_types=(
            [pltpu.VMEM((nch, _C), jnp.int32),
             pltpu.VMEM((_NBUF, _C, embed), jnp.float32),
             pltpu.VMEM((_NT, ntile, 8, _PITCH), jnp.float32)]
            + [pltpu.SemaphoreType.DMA] * (_NBUF + _NT)
        ),
        compiler_params=pltpu.CompilerParams(use_tc_tiling_on_sc=False,
                                             needs_layout_passes=False),
    )
    def k(q_hbm, table_hbm, out_hbm, idx_v, rows_v, gt_v, *sems):
        gsems, ssems = sems[:_NBUF], sems[_NBUF:]
        wid = lax.axis_index("s") * 2 + lax.axis_index("c")
        base = wid * per_w
        pltpu.sync_copy(q_hbm.at[wid], idx_v)

        iota = lax.iota(jnp.int32, _L)
        ei_vec = lax.bitwise_and(iota, 7)
        et_vecs = [lax.shift_right_logical(iota, 3) + 2 * kk
                   for kk in range(embed // _L)]

        def gather_start(i, b):
            return pltpu.async_copy(
                table_hbm.at[idx_v.at[i]], rows_v.at[b], gsems[b])

        def out_wait(t):
            pltpu.make_async_copy(
                gt_v.at[t, :, :, pl.ds(0, _C)],
                out_hbm.at[0, :, 0, :, :], ssems[t]).wait()

        for b in range(_NBUF):
            gather_start(b, b)

        def group(g, carry):
            for b in range(_NBUF):
                i = g * _NBUF + b
                f = base + i * _C
                s_idx = f // batch
                bt_idx = (f % batch) // _C
                t = b % _NT
                if b >= _NT:
                    out_wait(t)
                else:
                    @pl.when(g > 0)
                    def _():
                        out_wait(t)
                pltpu.make_async_copy(
                    table_hbm.at[idx_v.at[i]], rows_v.at[b], gsems[b]).wait()

                grow = rows_v.at[b]

                @plsc.parallel_loop(0, _C, unroll=16)
                def tr_body(bi):
                    bi_vec = jnp.full((_L,), 0, jnp.int32) + bi
                    for kk in range(embed // _L):
                        x = grow[bi, pl.ds(kk * _L, _L)]
                        plsc.store_scatter(
                            gt_v.at[t], [et_vecs[kk], ei_vec, bi_vec], x)

                pltpu.async_copy(
                    gt_v.at[t, :, :, pl.ds(0, _C)],
                    out_hbm.at[s_idx, :, bt_idx, :, :], ssems[t])

                @pl.when(g < ngrp - 1)
                def _():
                    gather_start(i + _NBUF, b)
            return carry

        lax.fori_loop(0, ngrp, group, 0)
        for t in range(_NT):
            out_wait(t)

    return k


def kernel(q, q_len, table):
    del q_len
    n_total = q.size
    batch, seq = q.shape
    vocab, embed = table.shape
    qt_blocked = q.T.reshape(_NW, n_total // (_NW * _C), _C).astype(jnp.int32)
    out = _make_gather(n_total, seq, batch, vocab, embed)(qt_blocked, table)
    return out.transpose(2, 4, 0, 1, 3).reshape(batch, seq, embed)
